# baseline (speedup 1.0000x reference)
import functools

import jax
import jax.numpy as jnp
from jax import lax
from jax.experimental import pallas as pl
from jax.experimental.pallas import tpu as pltpu
from jax.experimental.pallas import tpu_sc as plsc

_N = 512
_ALPHA = 0.02
_BETA = 0.1
_GAMMA = 0.02
_W1 = 0.5
_W2 = 0.5
_SHIFT = 0.0


def _pairwise_body(f_ref, aux_ref, out_ref):
    n = _N
    cf = f_ref.shape[1]

    row_i = jax.lax.broadcasted_iota(jnp.int32, (n, n), 0)
    col_i = jax.lax.broadcasted_iota(jnp.int32, (n, n), 1)
    eye = (row_i == col_i).astype(jnp.float32)
    ft = jax.lax.dot_general(eye, f_ref[0], (((1,), (1,)), ((), ())),
                             preferred_element_type=jnp.float32)
    auxt = jax.lax.dot_general(eye, aux_ref[0], (((1,), (1,)), ((), ())),
                               preferred_element_type=jnp.float32)

    lane_c = jax.lax.broadcasted_iota(jnp.int32, (1, cf), 1)
    lane_a = jax.lax.broadcasted_iota(jnp.int32, (1, 8), 1)

    def aux_col(c):
        onehot = (lane_a == c).astype(jnp.float32)
        return jnp.sum(auxt * onehot, axis=1, keepdims=True)

    def body(c, acc):
        row = f_ref[0, pl.ds(c, 1), :]
        onehot = (lane_c == c).astype(jnp.float32)
        col = jnp.sum(ft * onehot, axis=1, keepdims=True)
        d = col - row
        ad = jnp.abs(d)
        sl1 = jnp.where(ad < 1.0, 0.5 * d * d, ad - 0.5)
        return acc + sl1

    acc = jax.lax.fori_loop(0, cf, body, jnp.zeros((n, n), jnp.float32))
    feat_mean = acc * (1.0 / cf)

    def sqdiff(c):
        row = aux_ref[0, c:c + 1, :]
        col = aux_col(c)
        d = col - row
        return d * d

    gd = sqdiff(0) + sqdiff(1) + sqdiff(2)
    cd = sqdiff(3) + sqdiff(4)

    e1 = -(cd * (1.0 / (2.0 * _ALPHA)) + gd * (1.0 / (2.0 * _BETA)))
    e2 = -(cd * (1.0 / (2.0 * _GAMMA)))
    sim = (_W1 * jnp.exp(e1) + _W2 * jnp.exp(e2) - _SHIFT) * (1.0 / (_W1 + _W2))

    vprod = aux_col(5) * aux_ref[0, 5:6, :]
    sprod = aux_col(6) * aux_ref[0, 6:7, :]
    unc = jnp.sqrt(jnp.maximum(sprod, 1e-8))

    t = vprod * feat_mean * sim
    s_loss = jnp.sum(unc * t)
    s_raw = jnp.sum(t)
    s_v = jnp.sum(vprod)

    out_ref[0] = jnp.concatenate(
        [jnp.full((1, 128), s_loss, jnp.float32),
         jnp.full((1, 128), s_raw, jnp.float32),
         jnp.full((1, 128), s_v, jnp.float32)], axis=1)


def _pairwise_call(sel_feats, aux):
    b = sel_feats.shape[0]
    grid = (b,)
    return pl.pallas_call(
        _pairwise_body,
        grid=grid,
        in_specs=[
            pl.BlockSpec((1, sel_feats.shape[1], _N), lambda i: (i, 0, 0)),
            pl.BlockSpec((1, 8, _N), lambda i: (i, 0, 0)),
        ],
        out_specs=pl.BlockSpec((1, 1, 384), lambda i: (i, 0, 0)),
        out_shape=jax.ShapeDtypeStruct((b, 1, 384), jnp.float32),
    )(sel_feats, aux)


def _topk_thresh_body(v_ref, u_ref, mu_ref, thr_ref):
    v = v_ref[0]
    u = u_ref[0]
    g = -jnp.log(-jnp.log(u + 1e-20) + 1e-20)
    scores = jnp.log(v + 1e-8) + g
    s = jax.lax.bitcast_convert_type(scores, jnp.int32)
    m = jnp.where(s < 0, s ^ jnp.int32(0x7FFFFFFF), s)
    mu = jax.lax.bitcast_convert_type(m, jnp.uint32) ^ jnp.uint32(0x80000000)
    mu_ref[0] = mu

    def body(_, carry):
        lo, hi = carry
        mid = lo + (hi - lo) // jnp.uint32(2)
        cnt = jnp.sum((mu > mid).astype(jnp.int32))
        big = cnt >= _N
        return (jnp.where(big, mid + jnp.uint32(1), lo),
                jnp.where(big, hi, mid))

    lo0 = jnp.uint32(0)
    hi0 = jnp.uint32(0xFFFFFFFF)
    lo, hi = jax.lax.fori_loop(0, 33, body, (lo0, hi0))
    t = lo
    ngt = jnp.sum((mu > t).astype(jnp.int32)).astype(jnp.uint32)
    lane = jax.lax.broadcasted_iota(jnp.int32, (1, 128), 1)
    thr_ref[0] = jnp.where(lane == 0, t, jnp.where(lane == 1, ngt,
                                                   jnp.uint32(0)))


def _topk_select(valid3, u3, b, npix):
    mu, thr = pl.pallas_call(
        _topk_thresh_body,
        grid=(b,),
        in_specs=[
            pl.BlockSpec((1,) + valid3.shape[1:], lambda i: (i, 0, 0)),
            pl.BlockSpec((1,) + valid3.shape[1:], lambda i: (i, 0, 0)),
        ],
        out_specs=[
            pl.BlockSpec((1,) + valid3.shape[1:], lambda i: (i, 0, 0)),
            pl.BlockSpec((1, 1, 128), lambda i: (i, 0, 0)),
        ],
        out_shape=[jax.ShapeDtypeStruct(valid3.shape, jnp.uint32),
                   jax.ShapeDtypeStruct((b, 1, 128), jnp.uint32)],
    )(valid3, u3)
    t = thr[:, 0, 0][:, None]
    ngt = thr[:, 0, 1].astype(jnp.int32)
    muf = mu.reshape(b, npix)
    gt = muf > t
    eq = muf == t
    tie_rank = jnp.cumsum(eq.astype(jnp.int32), axis=1)
    need = (_N - ngt)[:, None]
    accept = gt | (eq & (tie_rank <= need))
    c = jnp.cumsum(accept.astype(jnp.int32), axis=1)
    dest = jnp.where(accept, c - 1, _N + 1)
    cols = jnp.broadcast_to(jnp.arange(npix, dtype=jnp.int32)[None, :],
                            (b, npix))
    rows = jnp.broadcast_to(jnp.arange(b, dtype=jnp.int32)[:, None], (b, npix))
    offs = jnp.zeros((b, _N + 1), jnp.int32).at[rows, dest].set(
        cols, mode="drop", unique_indices=True)
    return offs[:, :_N]


def _sc_gather(offs, feats1d, guid1d, valid1d, scales1d, b, cf, cg, hw, h, w):
    n = _N
    cs = 128
    chunks_per_b = n // cs
    chalf = cf // 2
    nsl = cs // 16
    mesh = plsc.VectorSubcoreMesh(core_axis_name="c", subcore_axis_name="s")

    @functools.partial(
        pl.kernel, mesh=mesh,
        out_type=[jax.ShapeDtypeStruct((b, cf, n), jnp.float32),
                  jax.ShapeDtypeStruct((b, 8, n), jnp.float32)],
        scratch_types=[
            pltpu.VMEM((cs,), jnp.int32),
            pltpu.VMEM((8, cs), jnp.int32),
            pltpu.VMEM((chalf, cs), jnp.float32),
            pltpu.VMEM((8, cs), jnp.float32),
            pltpu.SemaphoreType.DMA,
        ],
    )
    def k(offs_hbm, feats_hbm, guid_hbm, valid_hbm, scales_hbm,
          outf_hbm, outa_hbm, offs_v, idx_buf, valf, vala, sem):
        wid = lax.axis_index("s") * 2 + lax.axis_index("c")
        bi = wid // (2 * chunks_per_b)
        rem = wid % (2 * chunks_per_b)
        chunk = rem % chunks_per_b
        half = rem // chunks_per_b
        col0 = chunk * cs
        c0 = half * chalf
        pltpu.sync_copy(offs_hbm.at[pl.ds(bi * n + col0, cs)], offs_v)

        _ = half
        for s in range(nsl):
            sl = pl.ds(s * 16, 16)
            o = offs_v[sl]
            yi = lax.div(o, w)
            xi = o - yi * w
            vala[3, sl] = yi.astype(jnp.float32) * (1.0 / h)
            vala[4, sl] = xi.astype(jnp.float32) * (1.0 / w)
            vala[7, sl] = jnp.zeros((16,), jnp.float32)
            idx_buf[0, sl] = o + bi * hw
        cps = [pltpu.async_copy(valid_hbm.at[idx_buf.at[0]], vala.at[5], sem),
               pltpu.async_copy(scales_hbm.at[idx_buf.at[0]], vala.at[6], sem)]
        for j in range(cg):
            for s in range(nsl):
                sl = pl.ds(s * 16, 16)
                idx_buf[1 + j, sl] = offs_v[sl] + (bi * cg + j) * hw
            cps.append(
                pltpu.async_copy(guid_hbm.at[idx_buf.at[1 + j]],
                                 vala.at[j], sem))
        for cp in cps:
            cp.wait()
        pltpu.sync_copy(vala, outa_hbm.at[bi, :, pl.ds(col0, cs)])

        def grp(g, carry):
            for j in range(8):
                c = g * 8 + j
                for s in range(nsl):
                    sl = pl.ds(s * 16, 16)
                    idx_buf[j, sl] = offs_v[sl] + (bi * cf + c0 + c) * hw
            gcps = [pltpu.async_copy(feats_hbm.at[idx_buf.at[j]],
                                     valf.at[g * 8 + j], sem)
                    for j in range(8)]
            for cp in gcps:
                cp.wait()
            return carry

        lax.fori_loop(0, chalf // 8, grp, 0)

        pltpu.sync_copy(valf,
                        outf_hbm.at[bi, pl.ds(c0, chalf), pl.ds(col0, cs)])

    return k(offs.reshape(-1), feats1d, guid1d, valid1d, scales1d)


def kernel(guidance, features, valid_mask, loss_scales):
    b, cg, h, w = guidance.shape
    cf = features.shape[1]
    n = _N

    npix = h * w
    u = jax.random.uniform(jax.random.key(42), (b, npix), dtype=jnp.float32)
    valid3 = valid_mask.reshape(b, npix // 128, 128)
    u3 = u.reshape(b, npix // 128, 128)
    offs = _topk_select(valid3, u3, b, npix)

    hw = h * w
    sel_feats, aux = _sc_gather(
        offs, features.reshape(-1), guidance.reshape(-1),
        valid_mask.reshape(-1), loss_scales.reshape(-1), b, cf, cg, hw, h, w)

    part = _pairwise_call(sel_feats, aux)
    s_loss = part[:, 0, 0].sum()
    s_raw = part[:, 0, 128].sum()
    s_v = part[:, 0, 256].sum()

    div = jnp.maximum(s_v, (b * n * n) / 2.0)
    loss = s_loss / div
    raw_loss = jax.lax.stop_gradient(s_raw) / div
    return (loss, raw_loss)


# speedup vs baseline: 3.0253x; 3.0253x over previous
import functools

import jax
import jax.numpy as jnp
from jax import lax
from jax.experimental import pallas as pl
from jax.experimental.pallas import tpu as pltpu
from jax.experimental.pallas import tpu_sc as plsc

_N = 512
_ALPHA = 0.02
_BETA = 0.1
_GAMMA = 0.02
_W1 = 0.5
_W2 = 0.5
_SHIFT = 0.0


def _pairwise_body(f_ref, aux_ref, out_ref):
    n = _N
    cf = f_ref.shape[1]

    row_i = jax.lax.broadcasted_iota(jnp.int32, (n, n), 0)
    col_i = jax.lax.broadcasted_iota(jnp.int32, (n, n), 1)
    eye = (row_i == col_i).astype(jnp.float32)
    ft = jax.lax.dot_general(eye, f_ref[0], (((1,), (1,)), ((), ())),
                             preferred_element_type=jnp.float32)
    auxt = jax.lax.dot_general(eye, aux_ref[0], (((1,), (1,)), ((), ())),
                               preferred_element_type=jnp.float32)

    lane_c = jax.lax.broadcasted_iota(jnp.int32, (1, cf), 1)
    lane_a = jax.lax.broadcasted_iota(jnp.int32, (1, 8), 1)

    def aux_col(c):
        onehot = (lane_a == c).astype(jnp.float32)
        return jnp.sum(auxt * onehot, axis=1, keepdims=True)

    def body(c, acc):
        row = f_ref[0, pl.ds(c, 1), :]
        onehot = (lane_c == c).astype(jnp.float32)
        col = jnp.sum(ft * onehot, axis=1, keepdims=True)
        d = col - row
        ad = jnp.abs(d)
        sl1 = jnp.where(ad < 1.0, 0.5 * d * d, ad - 0.5)
        return acc + sl1

    acc = jax.lax.fori_loop(0, cf, body, jnp.zeros((n, n), jnp.float32))
    feat_mean = acc * (1.0 / cf)

    def sqdiff(c):
        row = aux_ref[0, c:c + 1, :]
        col = aux_col(c)
        d = col - row
        return d * d

    gd = sqdiff(0) + sqdiff(1) + sqdiff(2)
    cd = sqdiff(3) + sqdiff(4)

    e1 = -(cd * (1.0 / (2.0 * _ALPHA)) + gd * (1.0 / (2.0 * _BETA)))
    e2 = -(cd * (1.0 / (2.0 * _GAMMA)))
    sim = (_W1 * jnp.exp(e1) + _W2 * jnp.exp(e2) - _SHIFT) * (1.0 / (_W1 + _W2))

    vprod = aux_col(5) * aux_ref[0, 5:6, :]
    sprod = aux_col(6) * aux_ref[0, 6:7, :]
    unc = jnp.sqrt(jnp.maximum(sprod, 1e-8))

    t = vprod * feat_mean * sim
    s_loss = jnp.sum(unc * t)
    s_raw = jnp.sum(t)
    s_v = jnp.sum(vprod)

    out_ref[0] = jnp.concatenate(
        [jnp.full((1, 128), s_loss, jnp.float32),
         jnp.full((1, 128), s_raw, jnp.float32),
         jnp.full((1, 128), s_v, jnp.float32)], axis=1)


def _pairwise_call(sel_feats, aux):
    b = sel_feats.shape[0]
    grid = (b,)
    return pl.pallas_call(
        _pairwise_body,
        grid=grid,
        in_specs=[
            pl.BlockSpec((1, sel_feats.shape[1], _N), lambda i: (i, 0, 0)),
            pl.BlockSpec((1, 8, _N), lambda i: (i, 0, 0)),
        ],
        out_specs=pl.BlockSpec((1, 1, 384), lambda i: (i, 0, 0)),
        out_shape=jax.ShapeDtypeStruct((b, 1, 384), jnp.float32),
    )(sel_feats, aux)


def _topk_thresh_body(v_ref, u_ref, mu_ref, thr_ref):
    v = v_ref[0]
    u = u_ref[0]
    g = -jnp.log(-jnp.log(u + 1e-20) + 1e-20)
    scores = jnp.log(v + 1e-8) + g
    s = jax.lax.bitcast_convert_type(scores, jnp.int32)
    m = jnp.where(s < 0, s ^ jnp.int32(0x7FFFFFFF), s)
    mu = jax.lax.bitcast_convert_type(m, jnp.uint32) ^ jnp.uint32(0x80000000)
    mu_ref[0] = mu

    def body(_, carry):
        lo, hi = carry
        mid = lo + (hi - lo) // jnp.uint32(2)
        cnt = jnp.sum((mu > mid).astype(jnp.int32))
        big = cnt >= _N
        return (jnp.where(big, mid + jnp.uint32(1), lo),
                jnp.where(big, hi, mid))

    lo0 = jnp.uint32(0)
    hi0 = jnp.uint32(0xFFFFFFFF)
    lo, hi = jax.lax.fori_loop(0, 33, body, (lo0, hi0))
    t = lo
    ngt = jnp.sum((mu > t).astype(jnp.int32)).astype(jnp.uint32)
    lane = jax.lax.broadcasted_iota(jnp.int32, (1, 128), 1)
    thr_ref[0] = jnp.where(lane == 0, t, jnp.where(lane == 1, ngt,
                                                   jnp.uint32(0)))


def _topk_select(valid3, u3, b, npix):
    mu, thr = pl.pallas_call(
        _topk_thresh_body,
        grid=(b,),
        in_specs=[
            pl.BlockSpec((1,) + valid3.shape[1:], lambda i: (i, 0, 0)),
            pl.BlockSpec((1,) + valid3.shape[1:], lambda i: (i, 0, 0)),
        ],
        out_specs=[
            pl.BlockSpec((1,) + valid3.shape[1:], lambda i: (i, 0, 0)),
            pl.BlockSpec((1, 1, 128), lambda i: (i, 0, 0)),
        ],
        out_shape=[jax.ShapeDtypeStruct(valid3.shape, jnp.uint32),
                   jax.ShapeDtypeStruct((b, 1, 128), jnp.uint32)],
    )(valid3, u3)
    t = thr[:, 0, 0][:, None]
    ngt = thr[:, 0, 1].astype(jnp.int32)
    muf = mu.reshape(b, npix)
    gt = muf > t
    eq = muf == t
    tie_rank = jnp.cumsum(eq.astype(jnp.int32), axis=1)
    need = (_N - ngt)[:, None]
    accept = gt | (eq & (tie_rank <= need))
    c = jnp.cumsum(accept.astype(jnp.int32), axis=1)
    slots = jnp.arange(1, _N + 1, dtype=jnp.int32)
    offs = jax.vmap(
        lambda cr: jnp.searchsorted(cr, slots, side="left",
                                    method="compare_all"))(c)
    return offs.astype(jnp.int32)


def _sc_gather(offs, feats1d, guid1d, valid1d, scales1d, b, cf, cg, hw, h, w):
    n = _N
    cs = 128
    chunks_per_b = n // cs
    chalf = cf // 2
    nsl = cs // 16
    mesh = plsc.VectorSubcoreMesh(core_axis_name="c", subcore_axis_name="s")

    @functools.partial(
        pl.kernel, mesh=mesh,
        out_type=[jax.ShapeDtypeStruct((b, cf, n), jnp.float32),
                  jax.ShapeDtypeStruct((b, 8, n), jnp.float32)],
        scratch_types=[
            pltpu.VMEM((cs,), jnp.int32),
            pltpu.VMEM((8, cs), jnp.int32),
            pltpu.VMEM((chalf, cs), jnp.float32),
            pltpu.VMEM((8, cs), jnp.float32),
            pltpu.SemaphoreType.DMA,
        ],
    )
    def k(offs_hbm, feats_hbm, guid_hbm, valid_hbm, scales_hbm,
          outf_hbm, outa_hbm, offs_v, idx_buf, valf, vala, sem):
        wid = lax.axis_index("s") * 2 + lax.axis_index("c")
        bi = wid // (2 * chunks_per_b)
        rem = wid % (2 * chunks_per_b)
        chunk = rem % chunks_per_b
        half = rem // chunks_per_b
        col0 = chunk * cs
        c0 = half * chalf
        pltpu.sync_copy(offs_hbm.at[pl.ds(bi * n + col0, cs)], offs_v)

        _ = half
        for s in range(nsl):
            sl = pl.ds(s * 16, 16)
            o = offs_v[sl]
            yi = lax.div(o, w)
            xi = o - yi * w
            vala[3, sl] = yi.astype(jnp.float32) * (1.0 / h)
            vala[4, sl] = xi.astype(jnp.float32) * (1.0 / w)
            vala[7, sl] = jnp.zeros((16,), jnp.float32)
            idx_buf[0, sl] = o + bi * hw
        cps = [pltpu.async_copy(valid_hbm.at[idx_buf.at[0]], vala.at[5], sem),
               pltpu.async_copy(scales_hbm.at[idx_buf.at[0]], vala.at[6], sem)]
        for j in range(cg):
            for s in range(nsl):
                sl = pl.ds(s * 16, 16)
                idx_buf[1 + j, sl] = offs_v[sl] + (bi * cg + j) * hw
            cps.append(
                pltpu.async_copy(guid_hbm.at[idx_buf.at[1 + j]],
                                 vala.at[j], sem))
        for cp in cps:
            cp.wait()
        pltpu.sync_copy(vala, outa_hbm.at[bi, :, pl.ds(col0, cs)])

        def grp(g, carry):
            for j in range(8):
                c = g * 8 + j
                for s in range(nsl):
                    sl = pl.ds(s * 16, 16)
                    idx_buf[j, sl] = offs_v[sl] + (bi * cf + c0 + c) * hw
            gcps = [pltpu.async_copy(feats_hbm.at[idx_buf.at[j]],
                                     valf.at[g * 8 + j], sem)
                    for j in range(8)]
            for cp in gcps:
                cp.wait()
            return carry

        lax.fori_loop(0, chalf // 8, grp, 0)

        pltpu.sync_copy(valf,
                        outf_hbm.at[bi, pl.ds(c0, chalf), pl.ds(col0, cs)])

    return k(offs.reshape(-1), feats1d, guid1d, valid1d, scales1d)


def kernel(guidance, features, valid_mask, loss_scales):
    b, cg, h, w = guidance.shape
    cf = features.shape[1]
    n = _N

    npix = h * w
    u = jax.random.uniform(jax.random.key(42), (b, npix), dtype=jnp.float32)
    valid3 = valid_mask.reshape(b, npix // 128, 128)
    u3 = u.reshape(b, npix // 128, 128)
    offs = _topk_select(valid3, u3, b, npix)

    hw = h * w
    sel_feats, aux = _sc_gather(
        offs, features.reshape(-1), guidance.reshape(-1),
        valid_mask.reshape(-1), loss_scales.reshape(-1), b, cf, cg, hw, h, w)

    part = _pairwise_call(sel_feats, aux)
    s_loss = part[:, 0, 0].sum()
    s_raw = part[:, 0, 128].sum()
    s_v = part[:, 0, 256].sum()

    div = jnp.maximum(s_v, (b * n * n) / 2.0)
    loss = s_loss / div
    raw_loss = jax.lax.stop_gradient(s_raw) / div
    return (loss, raw_loss)


# speedup vs baseline: 3.0278x; 1.0008x over previous
import functools

import jax
import jax.numpy as jnp
from jax import lax
from jax.experimental import pallas as pl
from jax.experimental.pallas import tpu as pltpu
from jax.experimental.pallas import tpu_sc as plsc

_N = 512
_ALPHA = 0.02
_BETA = 0.1
_GAMMA = 0.02
_W1 = 0.5
_W2 = 0.5
_SHIFT = 0.0


def _pairwise_body(f_ref, aux_ref, out_ref):
    n = _N
    cf = f_ref.shape[1]

    row_i = jax.lax.broadcasted_iota(jnp.int32, (n, n), 0)
    col_i = jax.lax.broadcasted_iota(jnp.int32, (n, n), 1)
    eye = (row_i == col_i).astype(jnp.float32)
    ft = jax.lax.dot_general(eye, f_ref[0], (((1,), (1,)), ((), ())),
                             preferred_element_type=jnp.float32)
    auxt = jax.lax.dot_general(eye, aux_ref[0], (((1,), (1,)), ((), ())),
                               preferred_element_type=jnp.float32)

    lane_c = jax.lax.broadcasted_iota(jnp.int32, (1, cf), 1)
    lane_a = jax.lax.broadcasted_iota(jnp.int32, (1, 8), 1)

    def aux_col(c):
        onehot = (lane_a == c).astype(jnp.float32)
        return jnp.sum(auxt * onehot, axis=1, keepdims=True)

    def body(c, acc):
        row = f_ref[0, pl.ds(c, 1), :]
        onehot = (lane_c == c).astype(jnp.float32)
        col = jnp.sum(ft * onehot, axis=1, keepdims=True)
        d = col - row
        ad = jnp.abs(d)
        sl1 = jnp.where(ad < 1.0, 0.5 * d * d, ad - 0.5)
        return acc + sl1

    acc = jax.lax.fori_loop(0, cf, body, jnp.zeros((n, n), jnp.float32))
    feat_mean = acc * (1.0 / cf)

    def sqdiff(c):
        row = aux_ref[0, c:c + 1, :]
        col = aux_col(c)
        d = col - row
        return d * d

    gd = sqdiff(0) + sqdiff(1) + sqdiff(2)
    cd = sqdiff(3) + sqdiff(4)

    e1 = -(cd * (1.0 / (2.0 * _ALPHA)) + gd * (1.0 / (2.0 * _BETA)))
    e2 = -(cd * (1.0 / (2.0 * _GAMMA)))
    sim = (_W1 * jnp.exp(e1) + _W2 * jnp.exp(e2) - _SHIFT) * (1.0 / (_W1 + _W2))

    vprod = aux_col(5) * aux_ref[0, 5:6, :]
    sprod = aux_col(6) * aux_ref[0, 6:7, :]
    unc = jnp.sqrt(jnp.maximum(sprod, 1e-8))

    t = vprod * feat_mean * sim
    s_loss = jnp.sum(unc * t)
    s_raw = jnp.sum(t)
    s_v = jnp.sum(vprod)

    out_ref[0] = jnp.concatenate(
        [jnp.full((1, 128), s_loss, jnp.float32),
         jnp.full((1, 128), s_raw, jnp.float32),
         jnp.full((1, 128), s_v, jnp.float32)], axis=1)


def _pairwise_call(sel_feats, aux):
    b = sel_feats.shape[0]
    grid = (b,)
    return pl.pallas_call(
        _pairwise_body,
        grid=grid,
        in_specs=[
            pl.BlockSpec((1, sel_feats.shape[1], _N), lambda i: (i, 0, 0)),
            pl.BlockSpec((1, 8, _N), lambda i: (i, 0, 0)),
        ],
        out_specs=pl.BlockSpec((1, 1, 384), lambda i: (i, 0, 0)),
        out_shape=jax.ShapeDtypeStruct((b, 1, 384), jnp.float32),
    )(sel_feats, aux)


def _topk_thresh_body(v_ref, u_ref, mu_ref, thr_ref):
    v = v_ref[0]
    u = u_ref[0]
    g = -jnp.log(-jnp.log(u + 1e-20) + 1e-20)
    scores = jnp.log(v + 1e-8) + g
    s = jax.lax.bitcast_convert_type(scores, jnp.int32)
    m = jnp.where(s < 0, s ^ jnp.int32(0x7FFFFFFF), s)
    mu = jax.lax.bitcast_convert_type(m, jnp.uint32) ^ jnp.uint32(0x80000000)
    mu_ref[0] = mu

    def body(_, carry):
        lo, hi = carry
        mid = lo + (hi - lo) // jnp.uint32(2)
        cnt = jnp.sum((mu > mid).astype(jnp.int32))
        big = cnt >= _N
        return (jnp.where(big, mid + jnp.uint32(1), lo),
                jnp.where(big, hi, mid))

    lo0 = jnp.uint32(0)
    hi0 = jnp.uint32(0xFFFFFFFF)
    lo, hi = jax.lax.fori_loop(0, 33, body, (lo0, hi0))
    t = lo
    ngt = jnp.sum((mu > t).astype(jnp.int32)).astype(jnp.uint32)
    lane = jax.lax.broadcasted_iota(jnp.int32, (1, 128), 1)
    thr_ref[0] = jnp.where(lane == 0, t, jnp.where(lane == 1, ngt,
                                                   jnp.uint32(0)))


def _topk_select(valid3, u3, b, npix):
    mu, thr = pl.pallas_call(
        _topk_thresh_body,
        grid=(b,),
        in_specs=[
            pl.BlockSpec((1,) + valid3.shape[1:], lambda i: (i, 0, 0)),
            pl.BlockSpec((1,) + valid3.shape[1:], lambda i: (i, 0, 0)),
        ],
        out_specs=[
            pl.BlockSpec((1,) + valid3.shape[1:], lambda i: (i, 0, 0)),
            pl.BlockSpec((1, 1, 128), lambda i: (i, 0, 0)),
        ],
        out_shape=[jax.ShapeDtypeStruct(valid3.shape, jnp.uint32),
                   jax.ShapeDtypeStruct((b, 1, 128), jnp.uint32)],
    )(valid3, u3)
    t = thr[:, 0, 0][:, None]
    ngt = thr[:, 0, 1].astype(jnp.int32)
    muf = mu.reshape(b, npix)
    gt = muf > t
    eq = muf == t
    tie_rank = jnp.cumsum(eq.astype(jnp.int32), axis=1)
    need = (_N - ngt)[:, None]
    accept = gt | (eq & (tie_rank <= need))
    c = jnp.cumsum(accept.astype(jnp.int32), axis=1)
    slots = jnp.arange(1, _N + 1, dtype=jnp.int32)
    offs = jax.vmap(
        lambda cr: jnp.searchsorted(cr, slots, side="left",
                                    method="compare_all"))(c)
    return offs.astype(jnp.int32)


def _sc_gather(offs, feats1d, guid1d, valid1d, scales1d, b, cf, cg, hw, h, w):
    n = _N
    cs = 128
    chunks_per_b = n // cs
    chalf = cf // 2
    nsl = cs // 16
    mesh = plsc.VectorSubcoreMesh(core_axis_name="c", subcore_axis_name="s")

    @functools.partial(
        pl.kernel, mesh=mesh,
        out_type=[jax.ShapeDtypeStruct((b, cf, n), jnp.float32),
                  jax.ShapeDtypeStruct((b, 8, n), jnp.float32)],
        scratch_types=[
            pltpu.VMEM((cs,), jnp.int32),
            pltpu.VMEM((8, cs), jnp.int32),
            pltpu.VMEM((chalf, cs), jnp.float32),
            pltpu.VMEM((8, cs), jnp.float32),
            pltpu.SemaphoreType.DMA,
        ],
    )
    def k(offs_hbm, feats_hbm, guid_hbm, valid_hbm, scales_hbm,
          outf_hbm, outa_hbm, offs_v, idx_buf, valf, vala, sem):
        wid = lax.axis_index("s") * 2 + lax.axis_index("c")
        bi = wid // (2 * chunks_per_b)
        rem = wid % (2 * chunks_per_b)
        chunk = rem % chunks_per_b
        half = rem // chunks_per_b
        col0 = chunk * cs
        c0 = half * chalf
        pltpu.sync_copy(offs_hbm.at[pl.ds(bi * n + col0, cs)], offs_v)

        for s in range(nsl):
            sl = pl.ds(s * 16, 16)
            o = offs_v[sl]
            yi = lax.div(o, w)
            xi = o - yi * w
            vala[3, sl] = yi.astype(jnp.float32) * (1.0 / h)
            vala[4, sl] = xi.astype(jnp.float32) * (1.0 / w)
            vala[7, sl] = jnp.zeros((16,), jnp.float32)
            idx_buf[0, sl] = o + bi * hw
        cps = [pltpu.async_copy(valid_hbm.at[idx_buf.at[0]], vala.at[5], sem),
               pltpu.async_copy(scales_hbm.at[idx_buf.at[0]], vala.at[6], sem)]
        for j in range(cg):
            for s in range(nsl):
                sl = pl.ds(s * 16, 16)
                idx_buf[1 + j, sl] = offs_v[sl] + (bi * cg + j) * hw
            cps.append(
                pltpu.async_copy(guid_hbm.at[idx_buf.at[1 + j]],
                                 vala.at[j], sem))
        for cp in cps:
            cp.wait()
        pltpu.sync_copy(vala, outa_hbm.at[bi, :, pl.ds(col0, cs)])

        def grp(g, carry):
            for j in range(8):
                c = g * 8 + j
                for s in range(nsl):
                    sl = pl.ds(s * 16, 16)
                    idx_buf[j, sl] = offs_v[sl] + (bi * cf + c0 + c) * hw
            gcps = [pltpu.async_copy(feats_hbm.at[idx_buf.at[j]],
                                     valf.at[g * 8 + j], sem)
                    for j in range(8)]
            for cp in gcps:
                cp.wait()
            return carry

        lax.fori_loop(0, chalf // 8, grp, 0)

        pltpu.sync_copy(valf,
                        outf_hbm.at[bi, pl.ds(c0, chalf), pl.ds(col0, cs)])

    return k(offs.reshape(-1), feats1d, guid1d, valid1d, scales1d)


def kernel(guidance, features, valid_mask, loss_scales):
    b, cg, h, w = guidance.shape
    cf = features.shape[1]
    n = _N

    npix = h * w
    u = jax.random.uniform(jax.random.key(42), (b, npix), dtype=jnp.float32)
    valid3 = valid_mask.reshape(b, npix // 128, 128)
    u3 = u.reshape(b, npix // 128, 128)
    offs = _topk_select(valid3, u3, b, npix)

    hw = h * w
    sel_feats, aux = _sc_gather(
        offs, features.reshape(-1), guidance.reshape(-1),
        valid_mask.reshape(-1), loss_scales.reshape(-1), b, cf, cg, hw, h, w)

    part = _pairwise_call(sel_feats, aux)
    s_loss = part[:, 0, 0].sum()
    s_raw = part[:, 0, 128].sum()
    s_v = part[:, 0, 256].sum()

    div = jnp.maximum(s_v, (b * n * n) / 2.0)
    loss = s_loss / div
    raw_loss = jax.lax.stop_gradient(s_raw) / div
    return (loss, raw_loss)
